# baseline (speedup 1.0000x reference)
import functools

import jax
import jax.numpy as jnp
import numpy as np
from jax.experimental import pallas as pl
from jax.experimental.pallas import tpu as pltpu
from jax.experimental.pallas import tpu_sc as plsc

E = 1024
H = 16
Dh = 64
NQ = 8192
NK = 8192
L = 128
HW = E // 2
QW = E // 4

QB = 16
RB = QB * L
PROJ_ROWS = 512
GATHER_W = 128

_PERM_AB = np.empty(E, np.int64)
_j = np.arange(HW)
_PERM_AB[:HW] = (_j % 8) * Dh + _j // 8
_PERM_AB[HW:] = (8 + _j % 8) * Dh + _j // 8

_HI = -65536


def _proj_kernel(x_ref, wt_ref, b_ref, o_ref):
    acc = jnp.dot(x_ref[...], wt_ref[...], preferred_element_type=jnp.float32)
    o_ref[...] = acc + b_ref[...]


def _proj_pack_kernel(x_ref, wt_ref, b_ref, o_ref):
    acc = jnp.dot(x_ref[...], wt_ref[...], preferred_element_type=jnp.float32)
    acc = acc + b_ref[...]
    a16 = acc[:, :HW].astype(jnp.bfloat16).astype(jnp.float32)
    b16 = acc[:, HW:].astype(jnp.bfloat16).astype(jnp.float32)
    ia = jax.lax.bitcast_convert_type(a16, jnp.int32)
    ib = jax.lax.bitcast_convert_type(b16, jnp.int32)
    o_ref[...] = jax.lax.shift_right_logical(ia, 16) | (ib & _HI)


def _project(x, wt, b, pack):
    n = x.shape[0]
    body = _proj_pack_kernel if pack else _proj_kernel
    ow = HW if pack else E
    odt = jnp.int32 if pack else jnp.float32
    return pl.pallas_call(
        body,
        grid=(n // PROJ_ROWS,),
        in_specs=[
            pl.BlockSpec((PROJ_ROWS, E), lambda i: (i, 0)),
            pl.BlockSpec((E, E), lambda i: (0, 0)),
            pl.BlockSpec((1, E), lambda i: (0, 0)),
        ],
        out_specs=pl.BlockSpec((PROJ_ROWS, ow), lambda i: (i, 0)),
        out_shape=jax.ShapeDtypeStruct((n, ow), odt),
    )(x, wt, b)


def _gather_half(tbl, idx_row):
    mesh = plsc.VectorSubcoreMesh(core_axis_name="core", subcore_axis_name="subcore")
    n_idx = idx_row.shape[1]
    w32 = tbl.shape[1]

    @functools.partial(
        pl.kernel,
        out_type=jax.ShapeDtypeStruct((n_idx, w32), jnp.int32),
        mesh=mesh,
    )
    def gather_kernel(t_hbm, i_hbm, o_hbm):
        def body(i_vmem, o_vmem):
            pltpu.sync_copy(t_hbm.at[i_vmem.at[0]], o_vmem)

        pltpu.emit_pipeline(
            body,
            grid=(n_idx // GATHER_W,),
            in_specs=[pl.BlockSpec((1, GATHER_W), lambda i: (0, i))],
            out_specs=[pl.BlockSpec((GATHER_W, w32), lambda i: (i, 0))],
            core_axis_name=("core", "subcore"),
            dimension_semantics=(pltpu.PARALLEL,),
        )(i_hbm, o_hbm)

    return gather_kernel(tbl, idx_row)


def _lo(w):
    return jax.lax.bitcast_convert_type(jnp.left_shift(w, 16), jnp.float32)


def _hi(w):
    return jax.lax.bitcast_convert_type(w & _HI, jnp.float32)


def _attn_kernel(q_ref, kgl_ref, kgh_ref, vgl_ref, vgh_ref, wo_ref, bo_ref,
                 out_ref, attn_ref):
    lane8 = jax.lax.broadcasted_iota(jnp.int32, (8, HW), 1)
    row8 = jax.lax.broadcasted_iota(jnp.int32, (8, HW), 0)
    mask8 = (lane8 % 8 == row8).astype(jnp.float32)

    out_rows = []
    for b in range(QB):
        rows = pl.ds(b * L, L)
        kwl = kgl_ref[rows, :]
        kwh = kgh_ref[rows, :]
        qa = q_ref[b:b + 1, :HW]
        qb_ = q_ref[b:b + 1, HW:]
        ta = _lo(kwl) * qa[:, :QW] + _lo(kwh) * qa[:, QW:]
        tb = _hi(kwl) * qb_[:, :QW] + _hi(kwh) * qb_[:, QW:]
        w_half = QW // 2
        while w_half >= 8:
            ta = ta[:, :w_half] + ta[:, w_half:2 * w_half]
            tb = tb[:, :w_half] + tb[:, w_half:2 * w_half]
            w_half //= 2
        s = jnp.concatenate([ta, tb], axis=1)
        m = jnp.max(s, axis=0, keepdims=True)
        es = jnp.exp(s - m)
        denom = jnp.sum(es, axis=0, keepdims=True)
        w = es / denom
        wt = w.T
        attn_ref[b:b + 1, :] = jnp.sum(wt, axis=0, keepdims=True) / H
        vwl = vgl_ref[rows, :]
        vwh = vgh_ref[rows, :]
        va = jnp.concatenate([_lo(vwl), _lo(vwh)], axis=1)
        vb = jnp.concatenate([_hi(vwl), _hi(vwh)], axis=1)
        oa = jnp.dot(wt[:8, :], va, preferred_element_type=jnp.float32)
        ob = jnp.dot(wt[8:, :], vb, preferred_element_type=jnp.float32)
        out_rows.append(jnp.concatenate([
            jnp.sum(oa * mask8, axis=0, keepdims=True),
            jnp.sum(ob * mask8, axis=0, keepdims=True),
        ], axis=1))
    out_pre = jnp.concatenate(out_rows, axis=0)
    out_ref[...] = (
        jnp.dot(out_pre, wo_ref[...], preferred_element_type=jnp.float32)
        + bo_ref[...]
    )


def _attention(q, kgl, kgh, vgl, vgh, wo_t_perm, bo):
    nq = q.shape[0]
    return pl.pallas_call(
        _attn_kernel,
        grid=(nq // QB,),
        in_specs=[
            pl.BlockSpec((QB, E), lambda i: (i, 0)),
            pl.BlockSpec((RB, QW), lambda i: (i, 0)),
            pl.BlockSpec((RB, QW), lambda i: (i, 0)),
            pl.BlockSpec((RB, QW), lambda i: (i, 0)),
            pl.BlockSpec((RB, QW), lambda i: (i, 0)),
            pl.BlockSpec((E, E), lambda i: (0, 0)),
            pl.BlockSpec((1, E), lambda i: (0, 0)),
        ],
        out_specs=[
            pl.BlockSpec((QB, E), lambda i: (i, 0)),
            pl.BlockSpec((QB, L), lambda i: (i, 0)),
        ],
        out_shape=[
            jax.ShapeDtypeStruct((nq, E), jnp.float32),
            jax.ShapeDtypeStruct((nq, L), jnp.float32),
        ],
    )(q, kgl, kgh, vgl, vgh, wo_t_perm, bo)


def kernel(query, key, value, index_pair, in_proj_weight, in_proj_bias, out_proj_weight, out_proj_bias):
    scale = 1.0 / np.sqrt(Dh)
    perm = jnp.asarray(_PERM_AB)
    wq_t = (in_proj_weight[:E].T * scale)[:, perm]
    wk_t = in_proj_weight[E:2 * E].T[:, perm]
    wv_t = in_proj_weight[2 * E:].T[:, perm]
    bq = (in_proj_bias[:E] * scale)[perm].reshape(1, E)
    bk = in_proj_bias[E:2 * E][perm].reshape(1, E)
    bv = in_proj_bias[2 * E:][perm].reshape(1, E)
    wo_t_perm = out_proj_weight.T[perm, :]
    bo = out_proj_bias.reshape(1, E)

    q = _project(query, wq_t, bq, pack=False)
    kp = _project(key, wk_t, bk, pack=True)
    vp = _project(value, wv_t, bv, pack=True)

    idx_all = jnp.maximum(index_pair, 0).astype(jnp.int32).reshape(NQ, L)

    n_chunks = 8
    cq = NQ // n_chunks
    outs, attns = [], []
    for c in range(n_chunks):
        idx_row = idx_all[c * cq:(c + 1) * cq, :].reshape(1, cq * L)
        kgl = _gather_half(kp[:, :QW], idx_row)
        kgh = _gather_half(kp[:, QW:], idx_row)
        vgl = _gather_half(vp[:, :QW], idx_row)
        vgh = _gather_half(vp[:, QW:], idx_row)
        o, a = _attention(q[c * cq:(c + 1) * cq, :], kgl, kgh, vgl, vgh,
                          wo_t_perm, bo)
        outs.append(o)
        attns.append(a)
    return jnp.concatenate(outs, axis=0), jnp.concatenate(attns, axis=0)


# speedup vs baseline: 1.0034x; 1.0034x over previous
import functools

import jax
import jax.numpy as jnp
import numpy as np
from jax.experimental import pallas as pl
from jax.experimental.pallas import tpu as pltpu
from jax.experimental.pallas import tpu_sc as plsc

E = 1024
H = 16
Dh = 64
NQ = 8192
NK = 8192
L = 128
HW = E // 2
QW = E // 4

QB = 8
RB = QB * L
PROJ_ROWS = 512
GATHER_W = 128

_PERM_AB = np.empty(E, np.int64)
_j = np.arange(HW)
_PERM_AB[:HW] = (_j % 8) * Dh + _j // 8
_PERM_AB[HW:] = (8 + _j % 8) * Dh + _j // 8

_HI = -65536


def _proj_kernel(x_ref, wt_ref, b_ref, o_ref):
    acc = jnp.dot(x_ref[...], wt_ref[...], preferred_element_type=jnp.float32)
    o_ref[...] = acc + b_ref[...]


def _proj_pack_kernel(x_ref, wt_ref, b_ref, o_ref):
    acc = jnp.dot(x_ref[...], wt_ref[...], preferred_element_type=jnp.float32)
    acc = acc + b_ref[...]
    a16 = acc[:, :HW].astype(jnp.bfloat16).astype(jnp.float32)
    b16 = acc[:, HW:].astype(jnp.bfloat16).astype(jnp.float32)
    ia = jax.lax.bitcast_convert_type(a16, jnp.int32)
    ib = jax.lax.bitcast_convert_type(b16, jnp.int32)
    o_ref[...] = jax.lax.shift_right_logical(ia, 16) | (ib & _HI)


def _project(x, wt, b, pack):
    n = x.shape[0]
    body = _proj_pack_kernel if pack else _proj_kernel
    ow = HW if pack else E
    odt = jnp.int32 if pack else jnp.float32
    return pl.pallas_call(
        body,
        grid=(n // PROJ_ROWS,),
        in_specs=[
            pl.BlockSpec((PROJ_ROWS, E), lambda i: (i, 0)),
            pl.BlockSpec((E, E), lambda i: (0, 0)),
            pl.BlockSpec((1, E), lambda i: (0, 0)),
        ],
        out_specs=pl.BlockSpec((PROJ_ROWS, ow), lambda i: (i, 0)),
        out_shape=jax.ShapeDtypeStruct((n, ow), odt),
    )(x, wt, b)


def _gather_half(tbl, idx_row):
    mesh = plsc.VectorSubcoreMesh(core_axis_name="core", subcore_axis_name="subcore")
    n_idx = idx_row.shape[1]
    w32 = tbl.shape[1]

    @functools.partial(
        pl.kernel,
        out_type=jax.ShapeDtypeStruct((n_idx, w32), jnp.int32),
        mesh=mesh,
    )
    def gather_kernel(t_hbm, i_hbm, o_hbm):
        def body(i_vmem, o_vmem):
            pltpu.sync_copy(t_hbm.at[i_vmem.at[0]], o_vmem)

        pltpu.emit_pipeline(
            body,
            grid=(n_idx // GATHER_W,),
            in_specs=[pl.BlockSpec((1, GATHER_W), lambda i: (0, i))],
            out_specs=[pl.BlockSpec((GATHER_W, w32), lambda i: (i, 0))],
            core_axis_name=("core", "subcore"),
            dimension_semantics=(pltpu.PARALLEL,),
        )(i_hbm, o_hbm)

    return gather_kernel(tbl, idx_row)


def _lo(w):
    return jax.lax.bitcast_convert_type(jnp.left_shift(w, 16), jnp.float32)


def _hi(w):
    return jax.lax.bitcast_convert_type(w & _HI, jnp.float32)


def _attn_kernel(q_ref, kgl_ref, kgh_ref, vgl_ref, vgh_ref, wo_ref, bo_ref,
                 out_ref, attn_ref):
    lane8 = jax.lax.broadcasted_iota(jnp.int32, (8, HW), 1)
    row8 = jax.lax.broadcasted_iota(jnp.int32, (8, HW), 0)
    mask8 = (lane8 % 8 == row8).astype(jnp.float32)

    out_rows = []
    for b in range(QB):
        rows = pl.ds(b * L, L)
        kwl = kgl_ref[rows, :]
        kwh = kgh_ref[rows, :]
        qa = q_ref[b:b + 1, :HW]
        qb_ = q_ref[b:b + 1, HW:]
        ta = _lo(kwl) * qa[:, :QW] + _lo(kwh) * qa[:, QW:]
        tb = _hi(kwl) * qb_[:, :QW] + _hi(kwh) * qb_[:, QW:]
        w_half = QW // 2
        while w_half >= 8:
            ta = ta[:, :w_half] + ta[:, w_half:2 * w_half]
            tb = tb[:, :w_half] + tb[:, w_half:2 * w_half]
            w_half //= 2
        s = jnp.concatenate([ta, tb], axis=1)
        m = jnp.max(s, axis=0, keepdims=True)
        es = jnp.exp(s - m)
        denom = jnp.sum(es, axis=0, keepdims=True)
        w = es / denom
        wt = w.T
        attn_ref[b:b + 1, :] = jnp.sum(wt, axis=0, keepdims=True) / H
        vwl = vgl_ref[rows, :]
        vwh = vgh_ref[rows, :]
        va = jnp.concatenate([_lo(vwl), _lo(vwh)], axis=1)
        vb = jnp.concatenate([_hi(vwl), _hi(vwh)], axis=1)
        oa = jnp.dot(wt[:8, :], va, preferred_element_type=jnp.float32)
        ob = jnp.dot(wt[8:, :], vb, preferred_element_type=jnp.float32)
        out_rows.append(jnp.concatenate([
            jnp.sum(oa * mask8, axis=0, keepdims=True),
            jnp.sum(ob * mask8, axis=0, keepdims=True),
        ], axis=1))
    out_pre = jnp.concatenate(out_rows, axis=0)
    out_ref[...] = (
        jnp.dot(out_pre, wo_ref[...], preferred_element_type=jnp.float32)
        + bo_ref[...]
    )


def _attention(q, kgl, kgh, vgl, vgh, wo_t_perm, bo):
    nq = q.shape[0]
    return pl.pallas_call(
        _attn_kernel,
        grid=(nq // QB,),
        in_specs=[
            pl.BlockSpec((QB, E), lambda i: (i, 0)),
            pl.BlockSpec((RB, QW), lambda i: (i, 0)),
            pl.BlockSpec((RB, QW), lambda i: (i, 0)),
            pl.BlockSpec((RB, QW), lambda i: (i, 0)),
            pl.BlockSpec((RB, QW), lambda i: (i, 0)),
            pl.BlockSpec((E, E), lambda i: (0, 0)),
            pl.BlockSpec((1, E), lambda i: (0, 0)),
        ],
        out_specs=[
            pl.BlockSpec((QB, E), lambda i: (i, 0)),
            pl.BlockSpec((QB, L), lambda i: (i, 0)),
        ],
        out_shape=[
            jax.ShapeDtypeStruct((nq, E), jnp.float32),
            jax.ShapeDtypeStruct((nq, L), jnp.float32),
        ],
    )(q, kgl, kgh, vgl, vgh, wo_t_perm, bo)


def kernel(query, key, value, index_pair, in_proj_weight, in_proj_bias, out_proj_weight, out_proj_bias):
    scale = 1.0 / np.sqrt(Dh)
    perm = jnp.asarray(_PERM_AB)
    wq_t = (in_proj_weight[:E].T * scale)[:, perm]
    wk_t = in_proj_weight[E:2 * E].T[:, perm]
    wv_t = in_proj_weight[2 * E:].T[:, perm]
    bq = (in_proj_bias[:E] * scale)[perm].reshape(1, E)
    bk = in_proj_bias[E:2 * E][perm].reshape(1, E)
    bv = in_proj_bias[2 * E:][perm].reshape(1, E)
    wo_t_perm = out_proj_weight.T[perm, :]
    bo = out_proj_bias.reshape(1, E)

    q = _project(query, wq_t, bq, pack=False)
    kp = _project(key, wk_t, bk, pack=True)
    vp = _project(value, wv_t, bv, pack=True)

    idx_all = jnp.maximum(index_pair, 0).astype(jnp.int32).reshape(NQ, L)

    n_chunks = 8
    cq = NQ // n_chunks
    outs, attns = [], []
    for c in range(n_chunks):
        idx_row = idx_all[c * cq:(c + 1) * cq, :].reshape(1, cq * L)
        kgl = _gather_half(kp[:, :QW], idx_row)
        kgh = _gather_half(kp[:, QW:], idx_row)
        vgl = _gather_half(vp[:, :QW], idx_row)
        vgh = _gather_half(vp[:, QW:], idx_row)
        o, a = _attention(q[c * cq:(c + 1) * cq, :], kgl, kgh, vgl, vgh,
                          wo_t_perm, bo)
        outs.append(o)
        attns.append(a)
    return jnp.concatenate(outs, axis=0), jnp.concatenate(attns, axis=0)
